# baseline (speedup 1.0000x reference)
import functools

import jax
import jax.numpy as jnp
from jax.experimental import pallas as pl
from jax.experimental.pallas import tpu as pltpu

_B, _T, _N, _F = 2, 4, 1024, 158
_H_DIM = 256
_HEADS = 4
_C = _H_DIM // _HEADS


_L2E = 1.4426950408889634


def _head_scores(x, a_ref):
    cols = [jnp.sum(x[:, hd * _C:(hd + 1) * _C] * a_ref[hd:hd + 1, :],
                    axis=1, keepdims=True) for hd in range(_HEADS)]
    return jnp.concatenate(cols, axis=1)


def _gat_layer(h, maskf, ones_c, W_ref, a_src_mat_ref, a_dst_mat_ref):
    x = jnp.dot(h, W_ref[...], preferred_element_type=jnp.float32)
    asrc_all = _head_scores(x, a_src_mat_ref)
    adst_all = _head_scores(x, a_dst_mat_ref)
    amax = jnp.max(asrc_all, axis=0, keepdims=True)
    vmax = adst_all + amax
    m2 = jnp.maximum(vmax, 0.2 * vmax) * _L2E
    p_col = adst_all * _L2E - m2
    q_col = adst_all * (0.2 * _L2E) - m2
    p_row = (asrc_all * _L2E).T
    q_row = (asrc_all * (0.2 * _L2E)).T
    outs = []
    for hd in range(_HEADS):
        xh = x[:, hd * _C:(hd + 1) * _C]
        arg = jnp.maximum(p_col[:, hd:hd + 1] + p_row[hd:hd + 1, :],
                          q_col[:, hd:hd + 1] + q_row[hd:hd + 1, :])
        e = jnp.exp2(arg) * maskf
        res = jnp.dot(e, jnp.concatenate([xh, ones_c], axis=1),
                      preferred_element_type=jnp.float32)
        outs.append(res[:, :_C] * (1.0 / res[:, _C:_C + 1]))
    return outs


def _fused_kernel(xc_ref, sgt_ref, W_in_ref, b_in_ref, ln_g_ref, ln_b_ref,
                  W0_ref, a_src0_ref, a_dst0_ref, bias0_ref,
                  W1_ref, a_src1_ref, a_dst1_ref, bias1_ref,
                  W_out_ref, b_out_ref, out_ref):
    xc = xc_ref[0, 0]
    h = jnp.dot(xc, W_in_ref[...],
                preferred_element_type=jnp.float32) + b_in_ref[...]
    mu = jnp.mean(h, axis=-1, keepdims=True)
    var = jnp.mean((h - mu) * (h - mu), axis=-1, keepdims=True)
    h = (h - mu) * jax.lax.rsqrt(var + 1e-5) * ln_g_ref[...] + ln_b_ref[...]
    h = jnp.maximum(h, 0.0)

    sg = sgt_ref[0]
    row = jax.lax.broadcasted_iota(jnp.int32, (_N, _N), 0)
    col = jax.lax.broadcasted_iota(jnp.int32, (_N, _N), 1)
    m_src = jnp.where((sg != 0.0) | (row == col), 1.0, 0.0)
    maskf = m_src.T
    ones_c = jnp.ones((_N, _C), jnp.float32)

    o0 = _gat_layer(h, maskf, ones_c, W0_ref, a_src0_ref, a_dst0_ref)
    g0 = jnp.concatenate(o0, axis=1) + bias0_ref[...]
    g0 = jnp.where(g0 > 0.0, g0, jnp.exp(jnp.minimum(g0, 0.0)) - 1.0)
    h1 = h + g0

    o1 = _gat_layer(h1, maskf, ones_c, W1_ref, a_src1_ref, a_dst1_ref)
    agg = (o1[0] + o1[1] + o1[2] + o1[3]) * 0.25 + bias1_ref[...]
    out_ref[0] = jnp.dot(agg, W_out_ref[...],
                         preferred_element_type=jnp.float32) + b_out_ref[...]


@jax.jit
def kernel(x_alpha, sector_graph, W_in, b_in, ln_g, ln_b,
           W0, a_src0, a_dst0, bias0, W1, a_src1, a_dst1, bias1,
           W_out, b_out):
    full2d = lambda shape: pl.BlockSpec(shape, lambda b: (0, 0))
    grid_spec = pl.GridSpec(
        grid=(_B,),
        in_specs=[
            pl.BlockSpec((1, 1, _N, _F), lambda b: (b, _T - 1, 0, 0)),
            pl.BlockSpec((1, _N, _N), lambda b: (b, 0, 0)),
            full2d((_F, _H_DIM)),
            full2d((1, _H_DIM)),
            full2d((1, _H_DIM)),
            full2d((1, _H_DIM)),
            full2d((_H_DIM, _H_DIM)),
            full2d((_HEADS, _C)),
            full2d((_HEADS, _C)),
            full2d((1, _H_DIM)),
            full2d((_H_DIM, _H_DIM)),
            full2d((_HEADS, _C)),
            full2d((_HEADS, _C)),
            full2d((1, _C)),
            full2d((_C, _H_DIM)),
            full2d((1, _H_DIM)),
        ],
        out_specs=pl.BlockSpec((1, _N, _H_DIM), lambda b: (b, 0, 0)),
    )
    return pl.pallas_call(
        _fused_kernel,
        grid_spec=grid_spec,
        out_shape=jax.ShapeDtypeStruct((_B, _N, _H_DIM), jnp.float32),
        compiler_params=pltpu.CompilerParams(
            dimension_semantics=("arbitrary",),
        ),
    )(x_alpha, sector_graph, W_in, b_in.reshape(1, _H_DIM), ln_g.reshape(1, _H_DIM),
      ln_b.reshape(1, _H_DIM), W0, a_src0, a_dst0,
      bias0.reshape(1, _H_DIM), W1, a_src1, a_dst1,
      bias1.reshape(1, _C), W_out, b_out.reshape(1, _H_DIM))


# speedup vs baseline: 1.1363x; 1.1363x over previous
import functools

import jax
import jax.numpy as jnp
from jax.experimental import pallas as pl
from jax.experimental.pallas import tpu as pltpu

_B, _T, _N, _F = 2, 4, 1024, 158
_H_DIM = 256
_HEADS = 4
_C = _H_DIM // _HEADS


_L2E = 1.4426950408889634


def _fold_attn(W_ref, a_ref):
    cols = [jnp.sum(W_ref[:, hd * _C:(hd + 1) * _C] * a_ref[hd:hd + 1, :],
                    axis=1, keepdims=True) for hd in range(_HEADS)]
    return jnp.concatenate(cols, axis=1)


def _gat_layer(h, maskf, ones_c, W_ref, a_src_mat_ref, a_dst_mat_ref):
    x = jnp.dot(h, W_ref[...], preferred_element_type=jnp.float32)
    asrc_all = jnp.dot(h, _fold_attn(W_ref, a_src_mat_ref),
                       preferred_element_type=jnp.float32)
    adst_all = jnp.dot(h, _fold_attn(W_ref, a_dst_mat_ref),
                       preferred_element_type=jnp.float32)
    amax = jnp.max(asrc_all, axis=0, keepdims=True)
    vmax = adst_all + amax
    m2 = jnp.maximum(vmax, 0.2 * vmax) * _L2E
    p_col = adst_all * _L2E - m2
    q_col = adst_all * (0.2 * _L2E) - m2
    p_row = (asrc_all * _L2E).T
    q_row = (asrc_all * (0.2 * _L2E)).T
    outs = []
    for hd in range(_HEADS):
        xh = x[:, hd * _C:(hd + 1) * _C]
        arg = jnp.maximum(p_col[:, hd:hd + 1] + p_row[hd:hd + 1, :],
                          q_col[:, hd:hd + 1] + q_row[hd:hd + 1, :])
        e = jnp.exp2(arg) * maskf
        res = jnp.dot(e, jnp.concatenate([xh, ones_c], axis=1),
                      preferred_element_type=jnp.float32)
        outs.append(res[:, :_C] * (1.0 / res[:, _C:_C + 1]))
    return outs


def _fused_kernel(xc_ref, sgt_ref, W_in_ref, b_in_ref, ln_g_ref, ln_b_ref,
                  W0_ref, a_src0_ref, a_dst0_ref, bias0_ref,
                  W1_ref, a_src1_ref, a_dst1_ref, bias1_ref,
                  W_out_ref, b_out_ref, out_ref):
    xc = xc_ref[0, 0]
    h = jnp.dot(xc, W_in_ref[...],
                preferred_element_type=jnp.float32) + b_in_ref[...]
    mu = jnp.mean(h, axis=-1, keepdims=True)
    var = jnp.mean((h - mu) * (h - mu), axis=-1, keepdims=True)
    h = (h - mu) * jax.lax.rsqrt(var + 1e-5) * ln_g_ref[...] + ln_b_ref[...]
    h = jnp.maximum(h, 0.0)

    sg = sgt_ref[0]
    row = jax.lax.broadcasted_iota(jnp.int32, (_N, _N), 0)
    col = jax.lax.broadcasted_iota(jnp.int32, (_N, _N), 1)
    m_src = jnp.where((sg != 0.0) | (row == col), 1.0, 0.0)
    maskf = m_src.T
    ones_c = jnp.ones((_N, _C), jnp.float32)

    o0 = _gat_layer(h, maskf, ones_c, W0_ref, a_src0_ref, a_dst0_ref)
    g0 = jnp.concatenate(o0, axis=1) + bias0_ref[...]
    g0 = jnp.where(g0 > 0.0, g0, jnp.exp(jnp.minimum(g0, 0.0)) - 1.0)
    h1 = h + g0

    o1 = _gat_layer(h1, maskf, ones_c, W1_ref, a_src1_ref, a_dst1_ref)
    agg = (o1[0] + o1[1] + o1[2] + o1[3]) * 0.25 + bias1_ref[...]
    out_ref[0] = jnp.dot(agg, W_out_ref[...],
                         preferred_element_type=jnp.float32) + b_out_ref[...]


@jax.jit
def kernel(x_alpha, sector_graph, W_in, b_in, ln_g, ln_b,
           W0, a_src0, a_dst0, bias0, W1, a_src1, a_dst1, bias1,
           W_out, b_out):
    full2d = lambda shape: pl.BlockSpec(shape, lambda b: (0, 0))
    grid_spec = pl.GridSpec(
        grid=(_B,),
        in_specs=[
            pl.BlockSpec((1, 1, _N, _F), lambda b: (b, _T - 1, 0, 0)),
            pl.BlockSpec((1, _N, _N), lambda b: (b, 0, 0)),
            full2d((_F, _H_DIM)),
            full2d((1, _H_DIM)),
            full2d((1, _H_DIM)),
            full2d((1, _H_DIM)),
            full2d((_H_DIM, _H_DIM)),
            full2d((_HEADS, _C)),
            full2d((_HEADS, _C)),
            full2d((1, _H_DIM)),
            full2d((_H_DIM, _H_DIM)),
            full2d((_HEADS, _C)),
            full2d((_HEADS, _C)),
            full2d((1, _C)),
            full2d((_C, _H_DIM)),
            full2d((1, _H_DIM)),
        ],
        out_specs=pl.BlockSpec((1, _N, _H_DIM), lambda b: (b, 0, 0)),
    )
    return pl.pallas_call(
        _fused_kernel,
        grid_spec=grid_spec,
        out_shape=jax.ShapeDtypeStruct((_B, _N, _H_DIM), jnp.float32),
        compiler_params=pltpu.CompilerParams(
            dimension_semantics=("arbitrary",),
        ),
    )(x_alpha, sector_graph, W_in, b_in.reshape(1, _H_DIM), ln_g.reshape(1, _H_DIM),
      ln_b.reshape(1, _H_DIM), W0, a_src0, a_dst0,
      bias0.reshape(1, _H_DIM), W1, a_src1, a_dst1,
      bias1.reshape(1, _C), W_out, b_out.reshape(1, _H_DIM))


# speedup vs baseline: 1.4523x; 1.2781x over previous
import functools

import jax
import jax.numpy as jnp
from jax.experimental import pallas as pl
from jax.experimental.pallas import tpu as pltpu

_B, _T, _N, _F = 2, 4, 1024, 158
_H_DIM = 256
_HEADS = 4
_C = _H_DIM // _HEADS


_L2E = 1.4426950408889634


def _gat_layer(h, maskf, ones_c, W_ref, a_src_mat_ref, a_dst_mat_ref):
    x = jnp.dot(h, W_ref[...], preferred_element_type=jnp.float32)
    asrc_all = jnp.dot(x, a_src_mat_ref[...],
                       preferred_element_type=jnp.float32)
    adst_all = jnp.dot(x, a_dst_mat_ref[...],
                       preferred_element_type=jnp.float32)
    amax = jnp.max(asrc_all, axis=0, keepdims=True)
    vmax = adst_all + amax
    m2 = jnp.maximum(vmax, 0.2 * vmax) * _L2E
    p_col = adst_all * _L2E - m2
    q_col = adst_all * (0.2 * _L2E) - m2
    p_row = (asrc_all * _L2E).T
    q_row = (asrc_all * (0.2 * _L2E)).T
    outs = []
    for hd in range(_HEADS):
        xh = x[:, hd * _C:(hd + 1) * _C]
        arg = jnp.maximum(p_col[:, hd:hd + 1] + p_row[hd:hd + 1, :],
                          q_col[:, hd:hd + 1] + q_row[hd:hd + 1, :])
        e = jnp.exp2(arg) * maskf
        res = jnp.dot(e, jnp.concatenate([xh, ones_c], axis=1),
                      preferred_element_type=jnp.float32)
        outs.append(res[:, :_C] * (1.0 / res[:, _C:_C + 1]))
    return outs


def _fused_kernel(xc_ref, sgt_ref, W_in_ref, b_in_ref, ln_g_ref, ln_b_ref,
                  W0_ref, a_src0_ref, a_dst0_ref, bias0_ref,
                  W1_ref, a_src1_ref, a_dst1_ref, bias1_ref,
                  W_out_ref, b_out_ref, out_ref):
    xc = xc_ref[0]
    h = jnp.dot(xc, W_in_ref[...],
                preferred_element_type=jnp.float32) + b_in_ref[...]
    mu = jnp.mean(h, axis=-1, keepdims=True)
    var = jnp.mean((h - mu) * (h - mu), axis=-1, keepdims=True)
    h = (h - mu) * jax.lax.rsqrt(var + 1e-5) * ln_g_ref[...] + ln_b_ref[...]
    h = jnp.maximum(h, 0.0)

    sg = sgt_ref[0]
    row = jax.lax.broadcasted_iota(jnp.int32, (_N, _N), 0)
    col = jax.lax.broadcasted_iota(jnp.int32, (_N, _N), 1)
    m_src = jnp.where((sg != 0.0) | (row == col), 1.0, 0.0)
    maskf = m_src.T
    ones_c = jnp.ones((_N, _C), jnp.float32)

    o0 = _gat_layer(h, maskf, ones_c, W0_ref, a_src0_ref, a_dst0_ref)
    g0 = jnp.concatenate(o0, axis=1) + bias0_ref[...]
    g0 = jnp.where(g0 > 0.0, g0, jnp.exp(jnp.minimum(g0, 0.0)) - 1.0)
    h1 = h + g0

    o1 = _gat_layer(h1, maskf, ones_c, W1_ref, a_src1_ref, a_dst1_ref)
    agg = (o1[0] + o1[1] + o1[2] + o1[3]) * 0.25 + bias1_ref[...]
    out_ref[0] = jnp.dot(agg, W_out_ref[...],
                         preferred_element_type=jnp.float32) + b_out_ref[...]


@jax.jit
def kernel(x_alpha, sector_graph, W_in, b_in, ln_g, ln_b,
           W0, a_src0, a_dst0, bias0, W1, a_src1, a_dst1, bias1,
           W_out, b_out):
    xc = x_alpha[:, -1, :, :]
    eye_rep = jnp.repeat(jnp.eye(_HEADS, dtype=jnp.float32), _C, axis=0)
    bd = lambda a: a.reshape(_HEADS * _C, 1) * eye_rep
    full2d = lambda shape: pl.BlockSpec(shape, lambda b: (0, 0))
    grid_spec = pl.GridSpec(
        grid=(_B,),
        in_specs=[
            pl.BlockSpec((1, _N, _F), lambda b: (b, 0, 0)),
            pl.BlockSpec((1, _N, _N), lambda b: (b, 0, 0)),
            full2d((_F, _H_DIM)),
            full2d((1, _H_DIM)),
            full2d((1, _H_DIM)),
            full2d((1, _H_DIM)),
            full2d((_H_DIM, _H_DIM)),
            full2d((_H_DIM, _HEADS)),
            full2d((_H_DIM, _HEADS)),
            full2d((1, _H_DIM)),
            full2d((_H_DIM, _H_DIM)),
            full2d((_H_DIM, _HEADS)),
            full2d((_H_DIM, _HEADS)),
            full2d((1, _C)),
            full2d((_C, _H_DIM)),
            full2d((1, _H_DIM)),
        ],
        out_specs=pl.BlockSpec((1, _N, _H_DIM), lambda b: (b, 0, 0)),
    )
    return pl.pallas_call(
        _fused_kernel,
        grid_spec=grid_spec,
        out_shape=jax.ShapeDtypeStruct((_B, _N, _H_DIM), jnp.float32),
        compiler_params=pltpu.CompilerParams(
            dimension_semantics=("arbitrary",),
        ),
    )(xc, sector_graph, W_in, b_in.reshape(1, _H_DIM), ln_g.reshape(1, _H_DIM),
      ln_b.reshape(1, _H_DIM), W0, bd(a_src0), bd(a_dst0),
      bias0.reshape(1, _H_DIM), W1, bd(a_src1), bd(a_dst1),
      bias1.reshape(1, _C), W_out, b_out.reshape(1, _H_DIM))
